# baseline (speedup 1.0000x reference)
import functools

import jax
import jax.numpy as jnp
from jax import lax
from jax.experimental import pallas as pl
from jax.experimental.pallas import tpu as pltpu
from jax.experimental.pallas import tpu_sc as plsc

N_NODES = 10000
N_EDGES = 320000
D_FEAT = 128
OUT_DIM = 128

NC = 1
NS = 16
NW = 32
BLOCKS_PER_TILE = NW // (NC * NS)
CHUNK = 128
CHUNKS_PER_W = 80
GROUP = 8
NG = CHUNKS_PER_W // GROUP
EDGES_PER_W = CHUNKS_PER_W * CHUNK
E_PAD = NW * EDGES_PER_W
ACC_ROWS = 10240
ROWS_PER_TILE = ACC_ROWS // NS
DUMMY_ROW = N_NODES


def _run_block(h_hbm, sidx_hbm, didx_hbm, sidx, didx, rows, sems, semi,
               acc, wid):
    pltpu.sync_copy(didx_hbm.at[wid], didx)
    pltpu.sync_copy(sidx_hbm.at[wid, pl.ds(0, GROUP)], sidx.at[0])
    pltpu.async_copy(sidx_hbm.at[wid, pl.ds(GROUP, GROUP)], sidx.at[1], semi)

    for b in (0, 1):
        pltpu.async_copy(h_hbm.at[sidx.at[0, b]], rows.at[b], sems.at[b])

    def chunk_step(j, carry):
        pb = j % 2
        g = j // GROUP
        pltpu.make_async_copy(
            h_hbm.at[sidx.at[0, 0]], rows.at[pb], sems.at[pb]).wait()
        pltpu.sync_copy(rows.at[pb], acc.at[didx.at[j]], add=True)

        @pl.when((j % GROUP == GROUP - 2) & (j + 2 < CHUNKS_PER_W))
        def _():
            pltpu.make_async_copy(
                sidx_hbm.at[wid, pl.ds(0, GROUP)],
                sidx.at[(g + 1) % 2], semi).wait()

        @pl.when(j + 2 < CHUNKS_PER_W)
        def _():
            j2 = j + 2
            pltpu.async_copy(
                h_hbm.at[sidx.at[(j2 // GROUP) % 2, j2 % GROUP]],
                rows.at[pb], sems.at[pb])

        @pl.when((j % GROUP == GROUP - 1) & (g + 2 < NG))
        def _():
            pltpu.async_copy(
                sidx_hbm.at[wid, pl.ds((g + 2) * GROUP, GROUP)],
                sidx.at[g % 2], semi)
        return carry

    lax.fori_loop(0, CHUNKS_PER_W, chunk_step, 0)


def _sc_body(h_hbm, zeros_hbm, sidx_hbm, didx_hbm, out_hbm,
             sidx, didx, rows, acc, sems, semi):
    sid = lax.axis_index("s")

    base = sid * ROWS_PER_TILE
    pltpu.sync_copy(zeros_hbm, acc.at[pl.ds(base, ROWS_PER_TILE)])
    plsc.subcore_barrier()

    for r in range(BLOCKS_PER_TILE):
        _run_block(h_hbm, sidx_hbm, didx_hbm, sidx, didx, rows, sems, semi,
                   acc, sid * BLOCKS_PER_TILE + r)
    plsc.subcore_barrier()

    pltpu.sync_copy(acc.at[pl.ds(base, ROWS_PER_TILE)],
                    out_hbm.at[pl.ds(base, ROWS_PER_TILE)])


@functools.cache
def _sc_aggregate():
    return pl.kernel(
        _sc_body,
        out_type=jax.ShapeDtypeStruct((ACC_ROWS, D_FEAT), jnp.float32),
        mesh=plsc.VectorSubcoreMesh(core_axis_name="c", subcore_axis_name="s",
                                    num_cores=NC),
        scratch_types=[
            pltpu.VMEM((2, GROUP, CHUNK), jnp.int32),
            pltpu.VMEM((CHUNKS_PER_W, CHUNK), jnp.int32),
            pltpu.VMEM((2, CHUNK, D_FEAT), jnp.float32),
            pltpu.VMEM_SHARED((ACC_ROWS, D_FEAT), jnp.float32),
            pltpu.SemaphoreType.DMA((2,)),
            pltpu.SemaphoreType.DMA,
        ],
    )


def _mm_body(p_ref, w_ref, o_ref):
    o_ref[...] = jnp.dot(p_ref[...], w_ref[...],
                         preferred_element_type=jnp.float32,
                         precision=jax.lax.Precision.HIGHEST)


def _tc_matmul(partial, w_t):
    blk = 1000
    return pl.pallas_call(
        _mm_body,
        grid=(N_NODES // blk,),
        in_specs=[
            pl.BlockSpec((blk, D_FEAT), lambda i: (i, 0)),
            pl.BlockSpec((D_FEAT, OUT_DIM), lambda i: (0, 0)),
        ],
        out_specs=pl.BlockSpec((blk, OUT_DIM), lambda i: (i, 0)),
        out_shape=jax.ShapeDtypeStruct((N_NODES, OUT_DIM), jnp.float32),
    )(partial, w_t)


def kernel(h, edge_index, edge_types, W_attn, W_lin):
    src = edge_index[0].astype(jnp.int32)
    dst = edge_index[1].astype(jnp.int32)
    pad = E_PAD - N_EDGES
    src3 = jnp.concatenate(
        [src, jnp.zeros((pad,), jnp.int32)]).reshape(NW, CHUNKS_PER_W, CHUNK)
    dst3 = jnp.concatenate(
        [dst, jnp.full((pad,), DUMMY_ROW, jnp.int32)]).reshape(
            NW, CHUNKS_PER_W, CHUNK)
    zeros = jnp.zeros((ROWS_PER_TILE, D_FEAT), jnp.float32)

    partial = _sc_aggregate()(h, zeros, src3, dst3)
    return _tc_matmul(partial, W_lin.T)


# speedup vs baseline: 1.1384x; 1.1384x over previous
import functools

import jax
import jax.numpy as jnp
from jax import lax
from jax.experimental import pallas as pl
from jax.experimental.pallas import tpu as pltpu
from jax.experimental.pallas import tpu_sc as plsc

N_NODES = 10000
N_EDGES = 320000
D_FEAT = 128
OUT_DIM = 128

NC = 2
NS = 16
CHUNK = 128
N_CHUNKS = 2560
GROUP = 8
E_PAD = N_CHUNKS * CHUNK
ACC_ROWS = 10240
ROWS_PER_TILE = ACC_ROWS // NS
DUMMY_ROW = N_NODES

Q0 = 40
Q1 = 120


def _run_range(h_hbm, sidx_hbm, didx_hbm, sidxg, didxg, rows,
               sems, semis, semid, acc, chunk0, nchunks):
    ngroups = nchunks // GROUP

    pltpu.sync_copy(sidx_hbm.at[pl.ds(chunk0, GROUP)], sidxg.at[0])
    pltpu.sync_copy(didx_hbm.at[pl.ds(chunk0, GROUP)], didxg.at[0])
    pltpu.async_copy(
        sidx_hbm.at[pl.ds(chunk0 + GROUP, GROUP)], sidxg.at[1], semis)
    pltpu.async_copy(
        didx_hbm.at[pl.ds(chunk0 + GROUP, GROUP)], didxg.at[1], semid)

    for b in (0, 1):
        pltpu.async_copy(h_hbm.at[sidxg.at[0, b]], rows.at[b], sems.at[b])

    def chunk_step(j, carry):
        pb = j % 2
        g = j // GROUP
        pltpu.make_async_copy(
            h_hbm.at[sidxg.at[0, 0]], rows.at[pb], sems.at[pb]).wait()
        pltpu.sync_copy(rows.at[pb], acc.at[didxg.at[g % 2, j % GROUP]],
                        add=True)

        @pl.when((j % GROUP == GROUP - 2) & (j + 2 < nchunks))
        def _():
            pltpu.make_async_copy(
                sidx_hbm.at[pl.ds(chunk0, GROUP)],
                sidxg.at[(g + 1) % 2], semis).wait()
            pltpu.make_async_copy(
                didx_hbm.at[pl.ds(chunk0, GROUP)],
                didxg.at[(g + 1) % 2], semid).wait()

        @pl.when(j + 2 < nchunks)
        def _():
            j2 = j + 2
            pltpu.async_copy(
                h_hbm.at[sidxg.at[(j2 // GROUP) % 2, j2 % GROUP]],
                rows.at[pb], sems.at[pb])

        @pl.when((j % GROUP == GROUP - 1) & (g + 2 < ngroups))
        def _():
            nxt = chunk0 + (g + 2) * GROUP
            pltpu.async_copy(
                sidx_hbm.at[pl.ds(nxt, GROUP)], sidxg.at[g % 2], semis)
            pltpu.async_copy(
                didx_hbm.at[pl.ds(nxt, GROUP)], didxg.at[g % 2], semid)
        return carry

    lax.fori_loop(0, nchunks, chunk_step, 0)


def _sc_body(h_hbm, zeros_hbm, sidx_hbm, didx_hbm, out_hbm,
             sidxg, didxg, rows, acc, sems, semis, semid):
    cid = lax.axis_index("c")
    sid = lax.axis_index("s")

    base = sid * ROWS_PER_TILE
    pltpu.sync_copy(zeros_hbm, acc.at[pl.ds(base, ROWS_PER_TILE)])
    plsc.subcore_barrier()

    chunk0 = jnp.where(cid == 0, sid * Q0, NS * Q0 + sid * Q1)
    nchunks = jnp.where(cid == 0, Q0, Q1)
    _run_range(h_hbm, sidx_hbm, didx_hbm, sidxg, didxg, rows,
               sems, semis, semid, acc, chunk0, nchunks)
    plsc.subcore_barrier()

    pltpu.sync_copy(acc.at[pl.ds(base, ROWS_PER_TILE)],
                    out_hbm.at[pl.ds(cid * ACC_ROWS + base, ROWS_PER_TILE)])


@functools.cache
def _sc_aggregate():
    return pl.kernel(
        _sc_body,
        out_type=jax.ShapeDtypeStruct((NC * ACC_ROWS, D_FEAT), jnp.float32),
        mesh=plsc.VectorSubcoreMesh(core_axis_name="c", subcore_axis_name="s"),
        scratch_types=[
            pltpu.VMEM((2, GROUP, CHUNK), jnp.int32),
            pltpu.VMEM((2, GROUP, CHUNK), jnp.int32),
            pltpu.VMEM((2, CHUNK, D_FEAT), jnp.float32),
            pltpu.VMEM_SHARED((ACC_ROWS, D_FEAT), jnp.float32),
            pltpu.SemaphoreType.DMA((2,)),
            pltpu.SemaphoreType.DMA,
            pltpu.SemaphoreType.DMA,
        ],
    )


def _mm_body(p_ref, w_ref, o_ref):
    a = p_ref[0] + p_ref[1]
    o_ref[...] = jnp.dot(a, w_ref[...],
                         preferred_element_type=jnp.float32,
                         precision=jax.lax.Precision.HIGHEST)


def _tc_matmul(partial, w_t):
    blk = 1000
    return pl.pallas_call(
        _mm_body,
        grid=(N_NODES // blk,),
        in_specs=[
            pl.BlockSpec((2, blk, D_FEAT), lambda i: (0, i, 0)),
            pl.BlockSpec((D_FEAT, OUT_DIM), lambda i: (0, 0)),
        ],
        out_specs=pl.BlockSpec((blk, OUT_DIM), lambda i: (i, 0)),
        out_shape=jax.ShapeDtypeStruct((N_NODES, OUT_DIM), jnp.float32),
    )(partial, w_t)


def kernel(h, edge_index, edge_types, W_attn, W_lin):
    src = edge_index[0].astype(jnp.int32)
    dst = edge_index[1].astype(jnp.int32)
    pad = E_PAD - N_EDGES
    src2 = jnp.concatenate(
        [src, jnp.zeros((pad,), jnp.int32)]).reshape(N_CHUNKS, CHUNK)
    dst2 = jnp.concatenate(
        [dst, jnp.full((pad,), DUMMY_ROW, jnp.int32)]).reshape(N_CHUNKS, CHUNK)
    zeros = jnp.zeros((ROWS_PER_TILE, D_FEAT), jnp.float32)

    partial = _sc_aggregate()(h, zeros, src2, dst2)
    return _tc_matmul(partial.reshape(NC, ACC_ROWS, D_FEAT), W_lin.T)


# speedup vs baseline: 1.1999x; 1.0540x over previous
import functools

import jax
import jax.numpy as jnp
from jax import lax
from jax.experimental import pallas as pl
from jax.experimental.pallas import tpu as pltpu
from jax.experimental.pallas import tpu_sc as plsc

N_NODES = 10000
N_EDGES = 320000
D_FEAT = 128
OUT_DIM = 128

NC = 2
NS = 16
CHUNK = 128
N_CHUNKS = 2560
GROUP = 8
E_PAD = N_CHUNKS * CHUNK
ACC_ROWS = 10240
ROWS_PER_TILE = ACC_ROWS // NS
DUMMY_ROW = N_NODES

Q0 = 120
Q1 = 40


def _run_range(h_hbm, sidx_hbm, didx_hbm, sidxg, didxg, rows,
               sems, semis, semid, acc, chunk0, nchunks):
    ngroups = nchunks // GROUP

    pltpu.sync_copy(sidx_hbm.at[pl.ds(chunk0, GROUP)], sidxg.at[0])
    pltpu.sync_copy(didx_hbm.at[pl.ds(chunk0, GROUP)], didxg.at[0])
    pltpu.async_copy(
        sidx_hbm.at[pl.ds(chunk0 + GROUP, GROUP)], sidxg.at[1], semis)
    pltpu.async_copy(
        didx_hbm.at[pl.ds(chunk0 + GROUP, GROUP)], didxg.at[1], semid)

    for b in (0, 1):
        pltpu.async_copy(h_hbm.at[sidxg.at[0, b]], rows.at[b], sems.at[b])

    def chunk_step(j, carry):
        pb = j % 2
        g = j // GROUP
        pltpu.make_async_copy(
            h_hbm.at[sidxg.at[0, 0]], rows.at[pb], sems.at[pb]).wait()
        pltpu.sync_copy(rows.at[pb], acc.at[didxg.at[g % 2, j % GROUP]],
                        add=True)

        @pl.when((j % GROUP == GROUP - 2) & (j + 2 < nchunks))
        def _():
            pltpu.make_async_copy(
                sidx_hbm.at[pl.ds(chunk0, GROUP)],
                sidxg.at[(g + 1) % 2], semis).wait()
            pltpu.make_async_copy(
                didx_hbm.at[pl.ds(chunk0, GROUP)],
                didxg.at[(g + 1) % 2], semid).wait()

        @pl.when(j + 2 < nchunks)
        def _():
            j2 = j + 2
            pltpu.async_copy(
                h_hbm.at[sidxg.at[(j2 // GROUP) % 2, j2 % GROUP]],
                rows.at[pb], sems.at[pb])

        @pl.when((j % GROUP == GROUP - 1) & (g + 2 < ngroups))
        def _():
            nxt = chunk0 + (g + 2) * GROUP
            pltpu.async_copy(
                sidx_hbm.at[pl.ds(nxt, GROUP)], sidxg.at[g % 2], semis)
            pltpu.async_copy(
                didx_hbm.at[pl.ds(nxt, GROUP)], didxg.at[g % 2], semid)
        return carry

    lax.fori_loop(0, nchunks, chunk_step, 0)


def _sc_body(h_hbm, zeros_hbm, sidx_hbm, didx_hbm, out_hbm,
             sidxg, didxg, rows, acc, sems, semis, semid):
    cid = lax.axis_index("c")
    sid = lax.axis_index("s")

    base = sid * ROWS_PER_TILE
    pltpu.sync_copy(zeros_hbm, acc.at[pl.ds(base, ROWS_PER_TILE)])
    plsc.subcore_barrier()

    chunk0 = jnp.where(cid == 0, sid * Q0, NS * Q0 + sid * Q1)
    nchunks = jnp.where(cid == 0, Q0, Q1)
    _run_range(h_hbm, sidx_hbm, didx_hbm, sidxg, didxg, rows,
               sems, semis, semid, acc, chunk0, nchunks)
    plsc.subcore_barrier()

    pltpu.sync_copy(acc.at[pl.ds(base, ROWS_PER_TILE)],
                    out_hbm.at[pl.ds(cid * ACC_ROWS + base, ROWS_PER_TILE)])


@functools.cache
def _sc_aggregate():
    return pl.kernel(
        _sc_body,
        out_type=jax.ShapeDtypeStruct((NC * ACC_ROWS, D_FEAT), jnp.float32),
        mesh=plsc.VectorSubcoreMesh(core_axis_name="c", subcore_axis_name="s"),
        scratch_types=[
            pltpu.VMEM((2, GROUP, CHUNK), jnp.int32),
            pltpu.VMEM((2, GROUP, CHUNK), jnp.int32),
            pltpu.VMEM((2, CHUNK, D_FEAT), jnp.float32),
            pltpu.VMEM_SHARED((ACC_ROWS, D_FEAT), jnp.float32),
            pltpu.SemaphoreType.DMA((2,)),
            pltpu.SemaphoreType.DMA,
            pltpu.SemaphoreType.DMA,
        ],
    )


def _mm_body(p_ref, w_ref, o_ref):
    a = p_ref[0] + p_ref[1]
    o_ref[...] = jnp.dot(a, w_ref[...],
                         preferred_element_type=jnp.float32,
                         precision=jax.lax.Precision.HIGHEST)


def _tc_matmul(partial, w_t):
    blk = 1000
    return pl.pallas_call(
        _mm_body,
        grid=(N_NODES // blk,),
        in_specs=[
            pl.BlockSpec((2, blk, D_FEAT), lambda i: (0, i, 0)),
            pl.BlockSpec((D_FEAT, OUT_DIM), lambda i: (0, 0)),
        ],
        out_specs=pl.BlockSpec((blk, OUT_DIM), lambda i: (i, 0)),
        out_shape=jax.ShapeDtypeStruct((N_NODES, OUT_DIM), jnp.float32),
    )(partial, w_t)


def kernel(h, edge_index, edge_types, W_attn, W_lin):
    src = edge_index[0].astype(jnp.int32)
    dst = edge_index[1].astype(jnp.int32)
    pad = E_PAD - N_EDGES
    src2 = jnp.concatenate(
        [src, jnp.zeros((pad,), jnp.int32)]).reshape(N_CHUNKS, CHUNK)
    dst2 = jnp.concatenate(
        [dst, jnp.full((pad,), DUMMY_ROW, jnp.int32)]).reshape(N_CHUNKS, CHUNK)
    zeros = jnp.zeros((ROWS_PER_TILE, D_FEAT), jnp.float32)

    partial = _sc_aggregate()(h, zeros, src2, dst2)
    return _tc_matmul(partial.reshape(NC, ACC_ROWS, D_FEAT), W_lin.T)
